# baseline (speedup 1.0000x reference)
import dataclasses
import functools

import jax
import jax.numpy as jnp
from jax import lax
from jax.experimental import pallas as pl
from jax.experimental.pallas import tpu as pltpu
from jax.experimental.pallas import tpu_sc as plsc

DF = 128
DE = 16
K = 64
LYR = 2

NC = 2
NS = 16
LANES = 16
NW = NC * NS
CHUNK = 128
WTILES = 10


def _bf16_bits(x_f32):
    u = lax.bitcast_convert_type(x_f32, jnp.int32)
    return lax.shift_right_logical(
        u + 0x7FFF + (lax.shift_right_logical(u, 16) & 1), 16)


def _embed_tc(x, W, b):
    n = x.shape[0]

    def body(x_ref, w_ref, b_ref, o_ref):
        o_ref[...] = (
            jnp.dot(x_ref[...], w_ref[...], preferred_element_type=jnp.float32)
            + b_ref[...]
        )

    return pl.pallas_call(
        body,
        out_shape=jax.ShapeDtypeStruct((n, DF), jnp.float32),
    )(x, W, b.reshape(1, DF))


def _gates_tc(ea, W_gate, b_gate, e_real, e_pad):
    blk = 2048
    nblk = e_pad // blk
    last_in = (e_real - 1) // blk

    def body(ea_ref, wg_ref, bg_ref, o0_ref, o1_ref):
        ea = ea_ref[...]
        i = pl.program_id(0)
        r_iota = lax.broadcasted_iota(jnp.int32, (blk, 1), 0)
        live = i * blk + r_iota < e_real
        for l, o_ref in ((0, o0_ref), (1, o1_ref)):
            z = (
                jnp.dot(ea, wg_ref[l], preferred_element_type=jnp.float32)
                + bg_ref[l]
            )
            g = jnp.where(live, jax.nn.sigmoid(z), 0.0)
            word = _bf16_bits(g[:, :DF // 2]) | (
                _bf16_bits(g[:, DF // 2:]) << 16)
            o_ref[...] = jnp.concatenate(
                [word[:blk // 2], word[blk // 2:]], axis=1)

    out = jax.ShapeDtypeStruct((e_pad // 2, DF), jnp.int32)
    return pl.pallas_call(
        body,
        grid=(nblk,),
        in_specs=[
            pl.BlockSpec((blk, DE), lambda i: (jnp.minimum(i, last_in), 0)),
            pl.BlockSpec((LYR, DE, DF), lambda i: (0, 0, 0)),
            pl.BlockSpec((LYR, DF), lambda i: (0, 0)),
        ],
        out_specs=[
            pl.BlockSpec((blk // 2, DF), lambda i: (i, 0)),
            pl.BlockSpec((blk // 2, DF), lambda i: (i, 0)),
        ],
        out_shape=[out, out],
    )(ea, W_gate, b_gate)


def _chunk_order(v_pad, nb, cpt):
    v4 = v_pad.reshape(nb, 2, 16, 64)
    inter = jnp.stack([v4[:, 0], v4[:, 1]], axis=-1)
    return inter.reshape(NW, cpt, CHUNK)


def _post_tc(h, agg, W_self_l, W_msg_l, b_h_l, W_assign_l, b_assign_l,
             g_true_l):
    n = h.shape[0]

    def body(h_ref, agg_ref, ws_ref, wm_ref, bh_ref, wa_ref, ba_ref, gt_ref,
             hn_ref, s_ref, loss_ref):
        a = agg_ref[0] + agg_ref[1]
        hn = jnp.dot(h_ref[...], ws_ref[...], preferred_element_type=jnp.float32)
        hn = hn + jnp.dot(a, wm_ref[...], preferred_element_type=jnp.float32)
        hn = jnp.maximum(hn + bh_ref[...], 0.0)
        hn_ref[...] = hn
        logits = (
            jnp.dot(hn, wa_ref[...], preferred_element_type=jnp.float32)
            + ba_ref[...]
        )
        m = jnp.max(logits, axis=-1, keepdims=True)
        e = jnp.exp(logits - m)
        s = e / jnp.sum(e, axis=-1, keepdims=True)
        s_ref[...] = s
        d = s - gt_ref[...]
        loss_ref[...] = jnp.reshape(jnp.sum(d * d) * (1.0 / (n * K)), (1, 1))

    return pl.pallas_call(
        body,
        out_shape=(
            jax.ShapeDtypeStruct((n, DF), jnp.float32),
            jax.ShapeDtypeStruct((n, K), jnp.float32),
            jax.ShapeDtypeStruct((1, 1), jnp.float32),
        ),
    )(h, agg, W_self_l, W_msg_l, b_h_l.reshape(1, DF), W_assign_l,
      b_assign_l.reshape(1, K), g_true_l)


def _sc_edge_layer(h, gate_w, idx4, zeros_hbm, n_nodes, cpt):
    wrows = n_nodes // WTILES
    nquad = cpt // 4
    hc = CHUNK // 2
    mesh = plsc.VectorSubcoreMesh(core_axis_name="c", subcore_axis_name="s")
    cp = pltpu.CompilerParams()
    if "needs_layout_passes" in pltpu.CompilerParams.__dataclass_fields__:
        cp = dataclasses.replace(cp, needs_layout_passes=False)

    @functools.partial(
        pl.kernel,
        compiler_params=cp,
        out_type=jax.ShapeDtypeStruct((NC, n_nodes, DF), jnp.float32),
        mesh=mesh,
        scratch_types=[
            pltpu.VMEM((4, CHUNK), jnp.int32),
            pltpu.VMEM((4, CHUNK), jnp.int32),
            pltpu.VMEM((CHUNK, DF), jnp.float32),
            pltpu.VMEM((CHUNK, DF), jnp.float32),
            pltpu.VMEM((hc, DF), jnp.int32),
            pltpu.VMEM((hc, DF), jnp.int32),
            pltpu.VMEM_SHARED((n_nodes, DF), jnp.float32),
            pltpu.SemaphoreType.DMA,
            pltpu.SemaphoreType.DMA,
            pltpu.SemaphoreType.DMA,
            pltpu.SemaphoreType.DMA,
            pltpu.SemaphoreType.DMA,
            pltpu.SemaphoreType.DMA,
        ],
    )
    def sck(h_hbm, gate_hbm, idx_hbm, z_hbm, agg_hbm,
            i0, i1, r0, r1, g0, g1, agg_sh,
            sem_i0, sem_i1, sem_h0, sem_h1, sem_g0, sem_g1):
        c = lax.axis_index("c")
        s = lax.axis_index("s")
        w = c * NS + s

        @pl.when(s < WTILES)
        def _():
            pltpu.sync_copy(z_hbm,
                            agg_sh.at[pl.ds(s * wrows, wrows)])
        plsc.subcore_barrier()

        gbase = w * cpt * hc

        def issue(jc, ibuf, sslot, rb, gb, sem_h, sem_g):
            pltpu.make_async_copy(h_hbm.at[ibuf.at[sslot]], rb, sem_h).start()
            pltpu.make_async_copy(
                gate_hbm.at[pl.ds(gbase + jc * hc, hc)], gb, sem_g
            ).start()

        def consume(ibuf, dslot, rb, gb, sem_h, sem_g):
            pltpu.make_async_copy(h_hbm.at[pl.ds(0, CHUNK)], rb, sem_h).wait()
            pltpu.make_async_copy(gate_hbm.at[pl.ds(0, hc)], gb, sem_g).wait()

            @pl.loop(0, hc)
            def _(rp):
                for rr in range(2):
                    r = 2 * rp + rr
                    for t in range(DF // 32):
                        gv = plsc.bitcast(
                            gb[rp, pl.ds(rr * (DF // 2) + 16 * t, 16)],
                            jnp.bfloat16)
                        ga, gb_ = plsc.unpack(
                            gv, format=plsc.PackFormat.INTERLEAVED)
                        lo = (r, pl.ds(16 * t, LANES))
                        hi = (r, pl.ds(DF // 2 + 16 * t, LANES))
                        rb[lo] = rb[lo] * ga
                        rb[hi] = rb[hi] * gb_

            pltpu.sync_copy(rb, agg_sh.at[ibuf.at[dslot]], add=True)

        def start_idx(p, ibuf, sem):
            pltpu.make_async_copy(idx_hbm.at[w, p], ibuf, sem).start()

        def wait_idx(ibuf, sem):
            pltpu.make_async_copy(idx_hbm.at[w, 0], ibuf, sem).wait()

        pltpu.sync_copy(idx_hbm.at[w, 0], i0)
        issue(0, i0, 0, r0, g0, sem_h0, sem_g0)
        start_idx(1, i1, sem_i1)

        @pl.loop(0, nquad)
        def _(u):
            c0 = 4 * u
            wait_idx(i1, sem_i1)
            issue(c0 + 1, i0, 2, r1, g1, sem_h1, sem_g1)
            consume(i0, 1, r0, g0, sem_h0, sem_g0)
            issue(c0 + 2, i1, 0, r0, g0, sem_h0, sem_g0)
            consume(i0, 3, r1, g1, sem_h1, sem_g1)

            @pl.when(u < nquad - 1)
            def _():
                start_idx(2 * u + 2, i0, sem_i0)

            issue(c0 + 3, i1, 2, r1, g1, sem_h1, sem_g1)
            consume(i1, 1, r0, g0, sem_h0, sem_g0)

            @pl.when(u < nquad - 1)
            def _():
                wait_idx(i0, sem_i0)
                issue(c0 + 4, i0, 0, r0, g0, sem_h0, sem_g0)

            consume(i1, 3, r1, g1, sem_h1, sem_g1)

            @pl.when(u < nquad - 1)
            def _():
                start_idx(2 * u + 3, i1, sem_i1)

        plsc.subcore_barrier()

        @pl.when(s < WTILES)
        def _():
            pltpu.sync_copy(agg_sh.at[pl.ds(s * wrows, wrows)],
                            agg_hbm.at[c, pl.ds(s * wrows, wrows)])

    return sck(h, gate_w, idx4, zeros_hbm)


def kernel(x, edge_index, edge_attr, grouping_matrices_true, W_embed, b_embed,
           W_gate, b_gate, W_self, W_msg, b_h, W_assign, b_assign):
    n = x.shape[0]
    e = edge_index.shape[1]

    tile_edges = CHUNK * NW
    cpt = -(-(-(-e // tile_edges)) // 4) * 4
    e_pad = tile_edges * cpt
    pad = e_pad - e

    src = jnp.concatenate([edge_index[0], jnp.zeros((pad,), jnp.int32)])
    dst = jnp.concatenate(
        [edge_index[1], jnp.arange(pad, dtype=jnp.int32) % n])
    nb = e_pad // 2048
    src3 = _chunk_order(src, nb, cpt)
    dst3 = _chunk_order(dst, nb, cpt)
    idx4 = jnp.stack(
        [src3[:, 0::2], dst3[:, 0::2], src3[:, 1::2], dst3[:, 1::2]], axis=2)

    zeros_hbm = jnp.zeros((n // WTILES, DF), jnp.float32)

    gate0, gate1 = _gates_tc(edge_attr, W_gate, b_gate, e, e_pad)
    h0 = _embed_tc(x, W_embed, b_embed)

    agg0 = _sc_edge_layer(h0, gate0, idx4, zeros_hbm, n, cpt)
    h1, s0, l0 = _post_tc(h0, agg0, W_self[0], W_msg[0], b_h[0],
                          W_assign[0], b_assign[0],
                          grouping_matrices_true[0])
    agg1 = _sc_edge_layer(h1, gate1, idx4, zeros_hbm, n, cpt)
    h2, s1, l1 = _post_tc(h1, agg1, W_self[1], W_msg[1], b_h[1],
                          W_assign[1], b_assign[1],
                          grouping_matrices_true[1])

    return h2, jnp.stack([s0, s1]), jnp.stack([l0[0, 0], l1[0, 0]])


# speedup vs baseline: 1.2921x; 1.2921x over previous
import dataclasses
import functools

import jax
import jax.numpy as jnp
from jax import lax
from jax.experimental import pallas as pl
from jax.experimental.pallas import tpu as pltpu
from jax.experimental.pallas import tpu_sc as plsc

DF = 128
DE = 16
K = 64
LYR = 2

NC = 2
NS = 16
LANES = 16
NW = NC * NS
CHUNK = 128
WTILES = 10


def _bf16_bits(x_f32):
    u = lax.bitcast_convert_type(x_f32, jnp.int32)
    return lax.shift_right_logical(
        u + 0x7FFF + (lax.shift_right_logical(u, 16) & 1), 16)


def _embed_tc(x, W, b):
    n = x.shape[0]

    def body(x_ref, w_ref, b_ref, o_ref, ow_ref):
        h = (
            jnp.dot(x_ref[...], w_ref[...], preferred_element_type=jnp.float32)
            + b_ref[...]
        )
        o_ref[...] = h
        ow_ref[...] = _bf16_bits(h[:, :DF // 2]) | (
            _bf16_bits(h[:, DF // 2:]) << 16)

    return pl.pallas_call(
        body,
        out_shape=(
            jax.ShapeDtypeStruct((n, DF), jnp.float32),
            jax.ShapeDtypeStruct((n, DF // 2), jnp.int32),
        ),
    )(x, W, b.reshape(1, DF))


def _gates_tc(ea, W_gate, b_gate, e_real, e_pad):
    blk = 4096
    nblk = e_pad // blk
    last_in = (e_real - 1) // blk
    hb = blk // 4

    def body(ea_ref, wg_ref, bg_ref, o0_ref, o1_ref):
        ea = ea_ref[...]
        i = pl.program_id(0)
        r_iota = lax.broadcasted_iota(jnp.int32, (blk, 1), 0)
        live = i * blk + r_iota < e_real
        for l, o_ref in ((0, o0_ref), (1, o1_ref)):
            z = (
                jnp.dot(ea, wg_ref[l], preferred_element_type=jnp.float32)
                + bg_ref[l]
            )
            g = jnp.where(live, jax.nn.sigmoid(z), 0.0)
            word = _bf16_bits(g[:, :DF // 2]) | (
                _bf16_bits(g[:, DF // 2:]) << 16)
            o_ref[...] = jnp.concatenate(
                [jnp.concatenate([word[:hb], word[hb:2 * hb]], axis=1),
                 jnp.concatenate([word[2 * hb:3 * hb], word[3 * hb:]], axis=1)],
                axis=0)

    out = jax.ShapeDtypeStruct((e_pad // 2, DF), jnp.int32)
    return pl.pallas_call(
        body,
        grid=(nblk,),
        in_specs=[
            pl.BlockSpec((blk, DE), lambda i: (jnp.minimum(i, last_in), 0)),
            pl.BlockSpec((LYR, DE, DF), lambda i: (0, 0, 0)),
            pl.BlockSpec((LYR, DF), lambda i: (0, 0)),
        ],
        out_specs=[
            pl.BlockSpec((blk // 2, DF), lambda i: (i, 0)),
            pl.BlockSpec((blk // 2, DF), lambda i: (i, 0)),
        ],
        out_shape=[out, out],
    )(ea, W_gate, b_gate)


def _chunk_order(v_pad, nb, cpt):
    v4 = v_pad.reshape(nb, 2, 16, 64)
    inter = jnp.stack([v4[:, 0], v4[:, 1]], axis=-1)
    return inter.reshape(NW, cpt, CHUNK)


def _post_tc(h, agg, W_self_l, W_msg_l, b_h_l, W_assign_l, b_assign_l,
             g_true_l):
    n = h.shape[0]

    def body(h_ref, agg_ref, ws_ref, wm_ref, bh_ref, wa_ref, ba_ref, gt_ref,
             hn_ref, hw_ref, s_ref, loss_ref):
        a = agg_ref[0] + agg_ref[1]
        hn = jnp.dot(h_ref[...], ws_ref[...], preferred_element_type=jnp.float32)
        hn = hn + jnp.dot(a, wm_ref[...], preferred_element_type=jnp.float32)
        hn = jnp.maximum(hn + bh_ref[...], 0.0)
        hn_ref[...] = hn
        hw_ref[...] = _bf16_bits(hn[:, :DF // 2]) | (
            _bf16_bits(hn[:, DF // 2:]) << 16)
        logits = (
            jnp.dot(hn, wa_ref[...], preferred_element_type=jnp.float32)
            + ba_ref[...]
        )
        m = jnp.max(logits, axis=-1, keepdims=True)
        e = jnp.exp(logits - m)
        s = e / jnp.sum(e, axis=-1, keepdims=True)
        s_ref[...] = s
        d = s - gt_ref[...]
        loss_ref[...] = jnp.reshape(jnp.sum(d * d) * (1.0 / (n * K)), (1, 1))

    return pl.pallas_call(
        body,
        out_shape=(
            jax.ShapeDtypeStruct((n, DF), jnp.float32),
            jax.ShapeDtypeStruct((n, DF // 2), jnp.int32),
            jax.ShapeDtypeStruct((n, K), jnp.float32),
            jax.ShapeDtypeStruct((1, 1), jnp.float32),
        ),
    )(h, agg, W_self_l, W_msg_l, b_h_l.reshape(1, DF), W_assign_l,
      b_assign_l.reshape(1, K), g_true_l)


def _sc_edge_layer(h, gate_w, idx4, zeros_hbm, n_nodes, cpt):
    wrows = n_nodes // WTILES
    nquad = cpt // 4
    hc = CHUNK // 2
    mesh = plsc.VectorSubcoreMesh(core_axis_name="c", subcore_axis_name="s")
    cp = pltpu.CompilerParams()
    if "needs_layout_passes" in pltpu.CompilerParams.__dataclass_fields__:
        cp = dataclasses.replace(cp, needs_layout_passes=False,
                                 use_tc_tiling_on_sc=False)

    @functools.partial(
        pl.kernel,
        compiler_params=cp,
        out_type=jax.ShapeDtypeStruct((NC, n_nodes, DF), jnp.float32),
        mesh=mesh,
        scratch_types=[
            pltpu.VMEM((4, CHUNK), jnp.int32),
            pltpu.VMEM((4, CHUNK), jnp.int32),
            pltpu.VMEM((CHUNK, DF // 2), jnp.int32),
            pltpu.VMEM((CHUNK, DF // 2), jnp.int32),
            pltpu.VMEM((hc, DF), jnp.int32),
            pltpu.VMEM((hc, DF), jnp.int32),
            pltpu.VMEM((CHUNK, DF), jnp.float32),
            pltpu.VMEM_SHARED((n_nodes, DF), jnp.float32),
            pltpu.SemaphoreType.DMA,
            pltpu.SemaphoreType.DMA,
            pltpu.SemaphoreType.DMA,
            pltpu.SemaphoreType.DMA,
            pltpu.SemaphoreType.DMA,
            pltpu.SemaphoreType.DMA,
        ],
    )
    def sck(h_hbm, gate_hbm, idx_hbm, z_hbm, agg_hbm,
            i0, i1, r0, r1, g0, g1, msg, agg_sh,
            sem_i0, sem_i1, sem_h0, sem_h1, sem_g0, sem_g1):
        c = lax.axis_index("c")
        s = lax.axis_index("s")
        w = c * NS + s

        @pl.when(s < WTILES)
        def _():
            pltpu.sync_copy(z_hbm,
                            agg_sh.at[pl.ds(s * wrows, wrows)])
        plsc.subcore_barrier()

        gbase = w * cpt * hc

        def issue(jc, ibuf, sslot, rb, gb, sem_h, sem_g):
            pltpu.make_async_copy(h_hbm.at[ibuf.at[sslot]], rb, sem_h).start()
            pltpu.make_async_copy(
                gate_hbm.at[pl.ds(gbase + jc * hc, hc)], gb, sem_g
            ).start()

        def consume(ibuf, dslot, rb, gb, sem_h, sem_g):
            pltpu.make_async_copy(h_hbm.at[pl.ds(0, CHUNK)], rb, sem_h).wait()
            pltpu.make_async_copy(gate_hbm.at[pl.ds(0, hc)], gb, sem_g).wait()

            @pl.loop(0, hc)
            def _(rp):
                for rr in range(2):
                    r = 2 * rp + rr
                    for t in range(DF // 32):
                        hv = plsc.bitcast(rb[r, pl.ds(16 * t, 16)],
                                          jnp.bfloat16)
                        gv = plsc.bitcast(
                            gb[rp, pl.ds(rr * (DF // 2) + 16 * t, 16)],
                            jnp.bfloat16)
                        ha, hb2 = plsc.unpack(
                            hv, format=plsc.PackFormat.INTERLEAVED)
                        ga, gb_ = plsc.unpack(
                            gv, format=plsc.PackFormat.INTERLEAVED)
                        msg[r, pl.ds(16 * t, LANES)] = ha * ga
                        msg[r, pl.ds(DF // 2 + 16 * t, LANES)] = hb2 * gb_

            pltpu.sync_copy(msg, agg_sh.at[ibuf.at[dslot]], add=True)

        def start_idx(p, ibuf, sem):
            pltpu.make_async_copy(idx_hbm.at[w, p], ibuf, sem).start()

        def wait_idx(ibuf, sem):
            pltpu.make_async_copy(idx_hbm.at[w, 0], ibuf, sem).wait()

        pltpu.sync_copy(idx_hbm.at[w, 0], i0)
        issue(0, i0, 0, r0, g0, sem_h0, sem_g0)
        start_idx(1, i1, sem_i1)

        @pl.loop(0, nquad)
        def _(u):
            c0 = 4 * u
            wait_idx(i1, sem_i1)
            issue(c0 + 1, i0, 2, r1, g1, sem_h1, sem_g1)
            consume(i0, 1, r0, g0, sem_h0, sem_g0)
            issue(c0 + 2, i1, 0, r0, g0, sem_h0, sem_g0)
            consume(i0, 3, r1, g1, sem_h1, sem_g1)

            @pl.when(u < nquad - 1)
            def _():
                start_idx(2 * u + 2, i0, sem_i0)

            issue(c0 + 3, i1, 2, r1, g1, sem_h1, sem_g1)
            consume(i1, 1, r0, g0, sem_h0, sem_g0)

            @pl.when(u < nquad - 1)
            def _():
                wait_idx(i0, sem_i0)
                issue(c0 + 4, i0, 0, r0, g0, sem_h0, sem_g0)

            consume(i1, 3, r1, g1, sem_h1, sem_g1)

            @pl.when(u < nquad - 1)
            def _():
                start_idx(2 * u + 3, i1, sem_i1)

        plsc.subcore_barrier()

        @pl.when(s < WTILES)
        def _():
            pltpu.sync_copy(agg_sh.at[pl.ds(s * wrows, wrows)],
                            agg_hbm.at[c, pl.ds(s * wrows, wrows)])

    return sck(h, gate_w, idx4, zeros_hbm)


def kernel(x, edge_index, edge_attr, grouping_matrices_true, W_embed, b_embed,
           W_gate, b_gate, W_self, W_msg, b_h, W_assign, b_assign):
    n = x.shape[0]
    e = edge_index.shape[1]

    tile_edges = CHUNK * NW
    cpt = -(-(-(-e // tile_edges)) // 4) * 4
    e_pad = tile_edges * cpt
    pad = e_pad - e

    src = jnp.concatenate(
        [edge_index[0], jnp.arange(pad, dtype=jnp.int32) % n])
    dst = jnp.concatenate(
        [edge_index[1], jnp.arange(pad, dtype=jnp.int32) % n])
    nb = e_pad // 2048
    src3 = _chunk_order(src, nb, cpt)
    dst3 = _chunk_order(dst, nb, cpt)
    idx4 = jnp.stack(
        [src3[:, 0::2], dst3[:, 0::2], src3[:, 1::2], dst3[:, 1::2]], axis=2)

    zeros_hbm = jnp.zeros((n // WTILES, DF), jnp.float32)

    gate0, gate1 = _gates_tc(edge_attr, W_gate, b_gate, e, e_pad)
    h0, h0w = _embed_tc(x, W_embed, b_embed)

    agg0 = _sc_edge_layer(h0w, gate0, idx4, zeros_hbm, n, cpt)
    h1, h1w, s0, l0 = _post_tc(h0, agg0, W_self[0], W_msg[0], b_h[0],
                               W_assign[0], b_assign[0],
                               grouping_matrices_true[0])
    agg1 = _sc_edge_layer(h1w, gate1, idx4, zeros_hbm, n, cpt)
    h2, _, s1, l1 = _post_tc(h1, agg1, W_self[1], W_msg[1], b_h[1],
                             W_assign[1], b_assign[1],
                             grouping_matrices_true[1])

    return h2, jnp.stack([s0, s1]), jnp.stack([l0[0, 0], l1[0, 0]])
